# Initial kernel scaffold; baseline (speedup 1.0000x reference)
import jax
import jax.numpy as jnp
from jax.experimental import pallas as pl


def kernel(X, edge_idx, C):
    raise NotImplementedError("write your pallas kernel here")


# speedup vs baseline: 13.2398x; 13.2398x over previous
import jax
import jax.numpy as jnp
from jax import lax
from jax.experimental import pallas as pl
from jax.experimental.pallas import tpu as pltpu
from jax.experimental.pallas import tpu_sc as plsc

N = 100000
K = 16
D = 12
F = 48
SCALE = 0.1

NC = 2
NS = 16
NW = NC * NS
LANES = 16

NODES_PER_W = N // NW
CH = 48
ECH = CH * K
N_CHUNKS = 66
TRAILER_START = NODES_PER_W - CH


def _edge_body(xm_hbm, ei_hbm, edge_out, mask_out,
               idx_v, idx2_v, xi_v, rows_v, out_v, mask_v, msc_v, sem):
    w = lax.axis_index("s") * NC + lax.axis_index("c")
    node0 = w * NODES_PER_W

    lane = lax.iota(jnp.int32, LANES)
    pat_j = [(lane + v * LANES) % D for v in range(3)]
    pat_i = [3 * ((lane + v * LANES) // D) + (lane + v * LANES) % 3
             for v in range(3)]
    c12 = jnp.full((LANES,), D, jnp.int32)

    def chunk_body(ci, carry):
        n0 = node0 + jnp.where(ci < N_CHUNKS - 1, ci * CH, TRAILER_START)
        pltpu.sync_copy(ei_hbm.at[pl.ds(n0 * K, ECH)], idx_v)
        for v in range(CH // LANES):
            idx2_v[pl.ds(v * LANES, LANES)] = n0 + lane + v * LANES
        cp1 = pltpu.async_copy(xm_hbm.at[idx2_v], xi_v, sem)
        cp2 = pltpu.async_copy(xm_hbm.at[idx_v], rows_v, sem)
        cp1.wait()
        cp2.wait()

        def node_body(i, carry2):
            e0 = i * K
            spl_i = jnp.full((LANES,), i, jnp.int32)
            mi = plsc.load_gather(xi_v, [spl_i, c12])
            mj = plsc.load_gather(rows_v, [e0 + lane, c12])
            m = mi * mj
            mask_v[pl.ds(e0, LANES)] = m
            misc = mi * SCALE
            xi = [plsc.load_gather(xi_v, [spl_i, p]) for p in pat_i]
            for k in range(K):
                e = e0 + k
                spl_e = jnp.full((LANES,), e, jnp.int32)
                mk = misc * plsc.load_gather(rows_v, [spl_e, c12])
                for v in range(3):
                    xj = plsc.load_gather(rows_v, [spl_e, pat_j[v]])
                    out_v[pl.ds(e * F + v * LANES, LANES)] = mk * (xj - xi[v])
            return carry2

        lax.fori_loop(0, CH, node_body, 0)
        pltpu.sync_copy(out_v, edge_out.at[pl.ds(n0 * K * F, ECH * F)])
        pltpu.sync_copy(mask_v, mask_out.at[pl.ds(n0 * K, ECH)])
        return carry

    lax.fori_loop(0, N_CHUNKS, chunk_body, 0)


def _edge_call(xm, ei):
    mesh = plsc.VectorSubcoreMesh(core_axis_name="c", subcore_axis_name="s")
    f = pl.kernel(
        _edge_body,
        out_type=(
            jax.ShapeDtypeStruct((N * K * F,), jnp.float32),
            jax.ShapeDtypeStruct((N * K,), jnp.float32),
        ),
        mesh=mesh,
        compiler_params=pltpu.CompilerParams(use_tc_tiling_on_sc=False, needs_layout_passes=False),
        scratch_types=[
            pltpu.VMEM((ECH,), jnp.int32),
            pltpu.VMEM((CH,), jnp.int32),
            pltpu.VMEM((CH, 16), jnp.float32),
            pltpu.VMEM((ECH, 16), jnp.float32),
            pltpu.VMEM((ECH * F,), jnp.float32),
            pltpu.VMEM((ECH,), jnp.float32),
            pltpu.VMEM((LANES,), jnp.float32),
            pltpu.SemaphoreType.DMA,
        ],
    )
    return f(xm, ei)


def _node_tc_kernel(x_ref, c_ref, o_ref):
    m = (c_ref[...] > 0).astype(jnp.float32)
    o_ref[...] = SCALE * x_ref[...] * m


def _node_call(x_flat, c2):
    nb = 1000
    return pl.pallas_call(
        _node_tc_kernel,
        grid=(N // nb,),
        in_specs=[
            pl.BlockSpec((nb, D), lambda i: (i, 0)),
            pl.BlockSpec((nb, 1), lambda i: (i, 0)),
        ],
        out_specs=pl.BlockSpec((nb, D), lambda i: (i, 0)),
        out_shape=jax.ShapeDtypeStruct((N, D), jnp.float32),
    )(x_flat, c2)


@jax.jit
def kernel(X, edge_idx, C):
    x_flat = X.reshape(N, D)
    mask = (C.reshape(N) > 0).astype(jnp.float32)
    xm = jnp.concatenate(
        [x_flat, mask[:, None], jnp.zeros((N, 3), jnp.float32)], axis=1)
    ei = edge_idx.reshape(N * K)

    edge_flat, mask_flat = _edge_call(xm, ei)
    node_h = _node_call(x_flat, C.reshape(N, 1))

    return (node_h.reshape(1, N, D),
            edge_flat.reshape(1, N, K, F),
            mask_flat.reshape(1, N, K, 1))


# speedup vs baseline: 14.3397x; 1.0831x over previous
import jax
import jax.numpy as jnp
from jax import lax
from jax.experimental import pallas as pl
from jax.experimental.pallas import tpu as pltpu
from jax.experimental.pallas import tpu_sc as plsc

N = 100000
K = 16
D = 12
F = 48
SCALE = 0.1

NC = 2
NS = 16
NW = NC * NS
LANES = 16

NODES_PER_W = N // NW
CH = 48
ECH = CH * K
N_CHUNKS = 66
TRAILER_START = NODES_PER_W - CH


def _edge_body(xm_hbm, ei_hbm, edge_out, mask_out,
               idx_v0, idx_v1, idx2_v0, idx2_v1, xi_v0, xi_v1,
               rows_v0, rows_v1, out_v0, out_v1, mask_v0, mask_v1,
               sem_i0, sem_i1, sem_g0, sem_g1, sem_o0, sem_o1):
    idx_v = (idx_v0, idx_v1)
    idx2_v = (idx2_v0, idx2_v1)
    xi_v = (xi_v0, xi_v1)
    rows_v = (rows_v0, rows_v1)
    out_v = (out_v0, out_v1)
    mask_v = (mask_v0, mask_v1)
    sem_i = (sem_i0, sem_i1)
    sem_g = (sem_g0, sem_g1)
    sem_o = (sem_o0, sem_o1)

    w = lax.axis_index("s") * NC + lax.axis_index("c")
    node0 = w * NODES_PER_W

    lane = lax.iota(jnp.int32, LANES)
    pat_j = [(lane + v * LANES) % D for v in range(3)]
    pat_i = [3 * ((lane + v * LANES) // D) + (lane + v * LANES) % 3
             for v in range(3)]
    c12 = jnp.full((LANES,), D, jnp.int32)

    def n0_of(ci):
        return node0 + jnp.minimum(ci * CH, TRAILER_START)

    def issue_idx(ci, p):
        pltpu.async_copy(ei_hbm.at[pl.ds(n0_of(ci) * K, ECH)],
                         idx_v[p], sem_i[p])

    def wait_idx(p):
        pltpu.make_async_copy(ei_hbm.at[pl.ds(0, ECH)],
                              idx_v[p], sem_i[p]).wait()

    def issue_gathers(ci, p):
        n0 = n0_of(ci)
        for v in range(CH // LANES):
            idx2_v[p][pl.ds(v * LANES, LANES)] = n0 + lane + v * LANES
        pltpu.async_copy(xm_hbm.at[idx2_v[p]], xi_v[p], sem_g[p])
        pltpu.async_copy(xm_hbm.at[idx_v[p]], rows_v[p], sem_g[p])

    def wait_gathers(p):
        pltpu.make_async_copy(xm_hbm.at[idx2_v[p]], xi_v[p], sem_g[p]).wait()
        pltpu.make_async_copy(xm_hbm.at[idx_v[p]], rows_v[p], sem_g[p]).wait()

    def issue_out(ci, p):
        n0 = n0_of(ci)
        pltpu.async_copy(out_v[p], edge_out.at[pl.ds(n0 * K * F, ECH * F)],
                         sem_o[p])
        pltpu.async_copy(mask_v[p], mask_out.at[pl.ds(n0 * K, ECH)],
                         sem_o[p])

    def wait_out(p):
        pltpu.make_async_copy(out_v[p], edge_out.at[pl.ds(0, ECH * F)],
                              sem_o[p]).wait()
        pltpu.make_async_copy(mask_v[p], mask_out.at[pl.ds(0, ECH)],
                              sem_o[p]).wait()

    def compute(ci, p):
        rows, xi_t, out_t, mask_t = rows_v[p], xi_v[p], out_v[p], mask_v[p]

        def node_body(i, carry2):
            e0 = i * K
            spl_i = jnp.full((LANES,), i, jnp.int32)
            mi = plsc.load_gather(xi_t, [spl_i, c12])
            mj = plsc.load_gather(rows, [e0 + lane, c12])
            mask_t[pl.ds(e0, LANES)] = mi * mj
            misc = mi * SCALE
            xi = [plsc.load_gather(xi_t, [spl_i, pp]) for pp in pat_i]
            for k in range(K):
                e = e0 + k
                spl_e = jnp.full((LANES,), e, jnp.int32)
                mk = misc * plsc.load_gather(rows, [spl_e, c12])
                for v in range(3):
                    xj = plsc.load_gather(rows, [spl_e, pat_j[v]])
                    out_t[pl.ds(e * F + v * LANES, LANES)] = mk * (xj - xi[v])
            return carry2

        lax.fori_loop(0, CH, node_body, 0)

    issue_idx(0, 0)
    wait_idx(0)
    issue_gathers(0, 0)
    issue_idx(1, 1)

    def pair_body(t, carry):
        for b in range(2):
            ci = 2 * t + b
            wait_gathers(b)

            @pl.when(ci + 1 < N_CHUNKS)
            def _():
                wait_idx(1 - b)
                issue_gathers(ci + 1, 1 - b)

            @pl.when(ci + 2 < N_CHUNKS)
            def _():
                issue_idx(ci + 2, b)

            @pl.when(ci >= 2)
            def _():
                wait_out(b)

            compute(ci, b)
            issue_out(ci, b)
        return carry

    lax.fori_loop(0, N_CHUNKS // 2, pair_body, 0)
    wait_out(0)
    wait_out(1)


def _edge_call(xm, ei):
    mesh = plsc.VectorSubcoreMesh(core_axis_name="c", subcore_axis_name="s")
    f = pl.kernel(
        _edge_body,
        out_type=(
            jax.ShapeDtypeStruct((N * K * F,), jnp.float32),
            jax.ShapeDtypeStruct((N * K,), jnp.float32),
        ),
        mesh=mesh,
        compiler_params=pltpu.CompilerParams(
            use_tc_tiling_on_sc=False, needs_layout_passes=False),
        scratch_types=[
            pltpu.VMEM((ECH,), jnp.int32),
            pltpu.VMEM((ECH,), jnp.int32),
            pltpu.VMEM((CH,), jnp.int32),
            pltpu.VMEM((CH,), jnp.int32),
            pltpu.VMEM((CH, 16), jnp.float32),
            pltpu.VMEM((CH, 16), jnp.float32),
            pltpu.VMEM((ECH, 16), jnp.float32),
            pltpu.VMEM((ECH, 16), jnp.float32),
            pltpu.VMEM((ECH * F,), jnp.float32),
            pltpu.VMEM((ECH * F,), jnp.float32),
            pltpu.VMEM((ECH,), jnp.float32),
            pltpu.VMEM((ECH,), jnp.float32),
            pltpu.SemaphoreType.DMA,
            pltpu.SemaphoreType.DMA,
            pltpu.SemaphoreType.DMA,
            pltpu.SemaphoreType.DMA,
            pltpu.SemaphoreType.DMA,
            pltpu.SemaphoreType.DMA,
        ],
    )
    return f(xm, ei)


def _node_tc_kernel(x_ref, c_ref, o_ref):
    m = (c_ref[...] > 0).astype(jnp.float32)
    o_ref[...] = SCALE * x_ref[...] * m


def _node_call(x_flat, c2):
    nb = 1000
    return pl.pallas_call(
        _node_tc_kernel,
        grid=(N // nb,),
        in_specs=[
            pl.BlockSpec((nb, D), lambda i: (i, 0)),
            pl.BlockSpec((nb, 1), lambda i: (i, 0)),
        ],
        out_specs=pl.BlockSpec((nb, D), lambda i: (i, 0)),
        out_shape=jax.ShapeDtypeStruct((N, D), jnp.float32),
    )(x_flat, c2)


@jax.jit
def kernel(X, edge_idx, C):
    x_flat = X.reshape(N, D)
    mask = (C.reshape(N) > 0).astype(jnp.float32)
    xm = jnp.concatenate(
        [x_flat, mask[:, None], jnp.zeros((N, 3), jnp.float32)], axis=1)
    ei = edge_idx.reshape(N * K)

    edge_flat, mask_flat = _edge_call(xm, ei)
    node_h = _node_call(x_flat, C.reshape(N, 1))

    return (node_h.reshape(1, N, D),
            edge_flat.reshape(1, N, K, F),
            mask_flat.reshape(1, N, K, 1))


# speedup vs baseline: 18.1254x; 1.2640x over previous
import jax
import jax.numpy as jnp
from jax import lax
from jax.experimental import pallas as pl
from jax.experimental.pallas import tpu as pltpu
from jax.experimental.pallas import tpu_sc as plsc

N = 100000
K = 16
D = 12
F = 48
SCALE = 0.1

NC = 2
NS = 16
NW = NC * NS
LANES = 16

NODES_PER_W = N // NW
CH = 48
ECH = CH * K
N_CHUNKS = 66
TRAILER_START = NODES_PER_W - CH


def _edge_body(xm_hbm, ei_hbm, edge_out, mask_out,
               idx_v0, idx_v1, idx2_v0, idx2_v1, xi_v0, xi_v1,
               rows_v0, rows_v1, out_v0, out_v1, mask_v0, mask_v1,
               sem_i0, sem_i1, sem_g0, sem_g1, sem_o0, sem_o1):
    idx_v = (idx_v0, idx_v1)
    idx2_v = (idx2_v0, idx2_v1)
    xi_v = (xi_v0, xi_v1)
    rows_v = (rows_v0, rows_v1)
    out_v = (out_v0, out_v1)
    mask_v = (mask_v0, mask_v1)
    sem_i = (sem_i0, sem_i1)
    sem_g = (sem_g0, sem_g1)
    sem_o = (sem_o0, sem_o1)

    w = lax.axis_index("s") * NC + lax.axis_index("c")
    node0 = w * NODES_PER_W

    lane = lax.iota(jnp.int32, LANES)
    pat_j = [(lane + v * LANES) % D for v in range(3)]
    pat_i = [3 * ((lane + v * LANES) // D) + (lane + v * LANES) % 3
             for v in range(3)]
    c12 = jnp.full((LANES,), D, jnp.int32)

    def n0_of(ci):
        return node0 + jnp.minimum(ci * CH, TRAILER_START)

    def issue_idx(ci, p):
        pltpu.async_copy(ei_hbm.at[pl.ds(n0_of(ci) * K, ECH)],
                         idx_v[p], sem_i[p])

    def wait_idx(p):
        pltpu.make_async_copy(ei_hbm.at[pl.ds(0, ECH)],
                              idx_v[p], sem_i[p]).wait()

    def issue_gathers(ci, p):
        n0 = n0_of(ci)
        for v in range(CH // LANES):
            idx2_v[p][pl.ds(v * LANES, LANES)] = n0 + lane + v * LANES
        pltpu.async_copy(xm_hbm.at[idx2_v[p]], xi_v[p], sem_g[p])
        pltpu.async_copy(xm_hbm.at[idx_v[p]], rows_v[p], sem_g[p])

    def wait_gathers(p):
        pltpu.make_async_copy(xm_hbm.at[idx2_v[p]], xi_v[p], sem_g[p]).wait()
        pltpu.make_async_copy(xm_hbm.at[idx_v[p]], rows_v[p], sem_g[p]).wait()

    def issue_out(ci, p):
        n0 = n0_of(ci)
        pltpu.async_copy(out_v[p], edge_out.at[pl.ds(n0 * K * F, ECH * F)],
                         sem_o[p])
        pltpu.async_copy(mask_v[p], mask_out.at[pl.ds(n0 * K, ECH)],
                         sem_o[p])

    def wait_out(p):
        pltpu.make_async_copy(out_v[p], edge_out.at[pl.ds(0, ECH * F)],
                              sem_o[p]).wait()
        pltpu.make_async_copy(mask_v[p], mask_out.at[pl.ds(0, ECH)],
                              sem_o[p]).wait()

    lane48 = lane * F

    def compute(ci, p):
        rows, xi_t, out_t, mask_t = rows_v[p], xi_v[p], out_v[p], mask_v[p]

        def node_body(i, carry2):
            e0 = i * K
            spl_i = jnp.full((LANES,), i, jnp.int32)
            erow = e0 + lane
            mi = plsc.load_gather(xi_t, [spl_i, c12])
            mj = plsc.load_gather(rows, [erow, c12])
            m = mi * mj
            mask_t[pl.ds(e0, LANES)] = m
            m01 = m * SCALE
            xj = [m01 * plsc.load_gather(
                      rows, [erow, jnp.full((LANES,), c, jnp.int32)])
                  for c in range(D)]
            ti = [m01 * plsc.load_gather(
                      xi_t, [spl_i, jnp.full((LANES,), c, jnp.int32)])
                  for c in range(D)]
            base = jnp.full((LANES,), e0 * F, jnp.int32) + lane48
            for o in range(F):
                val = xj[o % D] - ti[3 * (o // D) + o % 3]
                plsc.store_scatter(out_t, [base + o], val)
            return carry2

        lax.fori_loop(0, CH, node_body, 0)

    issue_idx(0, 0)
    wait_idx(0)
    issue_gathers(0, 0)
    issue_idx(1, 1)

    def pair_body(t, carry):
        for b in range(2):
            ci = 2 * t + b
            wait_gathers(b)

            @pl.when(ci + 1 < N_CHUNKS)
            def _():
                wait_idx(1 - b)
                issue_gathers(ci + 1, 1 - b)

            @pl.when(ci + 2 < N_CHUNKS)
            def _():
                issue_idx(ci + 2, b)

            @pl.when(ci >= 2)
            def _():
                wait_out(b)

            compute(ci, b)
            issue_out(ci, b)
        return carry

    lax.fori_loop(0, N_CHUNKS // 2, pair_body, 0)
    wait_out(0)
    wait_out(1)


def _edge_call(xm, ei):
    mesh = plsc.VectorSubcoreMesh(core_axis_name="c", subcore_axis_name="s")
    f = pl.kernel(
        _edge_body,
        out_type=(
            jax.ShapeDtypeStruct((N * K * F,), jnp.float32),
            jax.ShapeDtypeStruct((N * K,), jnp.float32),
        ),
        mesh=mesh,
        compiler_params=pltpu.CompilerParams(
            use_tc_tiling_on_sc=False, needs_layout_passes=False),
        scratch_types=[
            pltpu.VMEM((ECH,), jnp.int32),
            pltpu.VMEM((ECH,), jnp.int32),
            pltpu.VMEM((CH,), jnp.int32),
            pltpu.VMEM((CH,), jnp.int32),
            pltpu.VMEM((CH, 16), jnp.float32),
            pltpu.VMEM((CH, 16), jnp.float32),
            pltpu.VMEM((ECH, 16), jnp.float32),
            pltpu.VMEM((ECH, 16), jnp.float32),
            pltpu.VMEM((ECH * F,), jnp.float32),
            pltpu.VMEM((ECH * F,), jnp.float32),
            pltpu.VMEM((ECH,), jnp.float32),
            pltpu.VMEM((ECH,), jnp.float32),
            pltpu.SemaphoreType.DMA,
            pltpu.SemaphoreType.DMA,
            pltpu.SemaphoreType.DMA,
            pltpu.SemaphoreType.DMA,
            pltpu.SemaphoreType.DMA,
            pltpu.SemaphoreType.DMA,
        ],
    )
    return f(xm, ei)


def _node_tc_kernel(x_ref, c_ref, o_ref):
    m = (c_ref[...] > 0).astype(jnp.float32)
    o_ref[...] = SCALE * x_ref[...] * m


def _node_call(x_flat, c2):
    nb = 1000
    return pl.pallas_call(
        _node_tc_kernel,
        grid=(N // nb,),
        in_specs=[
            pl.BlockSpec((nb, D), lambda i: (i, 0)),
            pl.BlockSpec((nb, 1), lambda i: (i, 0)),
        ],
        out_specs=pl.BlockSpec((nb, D), lambda i: (i, 0)),
        out_shape=jax.ShapeDtypeStruct((N, D), jnp.float32),
    )(x_flat, c2)


@jax.jit
def kernel(X, edge_idx, C):
    x_flat = X.reshape(N, D)
    mask = (C.reshape(N) > 0).astype(jnp.float32)
    xm = jnp.concatenate(
        [x_flat, mask[:, None], jnp.zeros((N, 3), jnp.float32)], axis=1)
    ei = edge_idx.reshape(N * K)

    edge_flat, mask_flat = _edge_call(xm, ei)
    node_h = _node_call(x_flat, C.reshape(N, 1))

    return (node_h.reshape(1, N, D),
            edge_flat.reshape(1, N, K, F),
            mask_flat.reshape(1, N, K, 1))


# speedup vs baseline: 42.0852x; 2.3219x over previous
import jax
import jax.numpy as jnp
from jax import lax
from jax.experimental import pallas as pl
from jax.experimental.pallas import tpu as pltpu
from jax.experimental.pallas import tpu_sc as plsc

N = 100000
K = 16
D = 12
F = 48
SCALE = 0.1

NC = 2
NS = 16
NW = NC * NS
LANES = 16

NODES_PER_W = N // NW
CH = 48
ECH = CH * K
N_CHUNKS = 66
TRAILER_START = NODES_PER_W - CH


def _edge_body(xm_hbm, ei_hbm, edge_out, mask_out, node_out,
               idx_v0, idx_v1, idx2_v0, idx2_v1, xi_v0, xi_v1,
               rows_v0, rows_v1, out_v0, out_v1, mask_v0, mask_v1,
               node_v0, node_v1,
               sem_i0, sem_i1, sem_g0, sem_g1, sem_o0, sem_o1):
    idx_v = (idx_v0, idx_v1)
    idx2_v = (idx2_v0, idx2_v1)
    xi_v = (xi_v0, xi_v1)
    rows_v = (rows_v0, rows_v1)
    out_v = (out_v0, out_v1)
    mask_v = (mask_v0, mask_v1)
    node_v = (node_v0, node_v1)
    sem_i = (sem_i0, sem_i1)
    sem_g = (sem_g0, sem_g1)
    sem_o = (sem_o0, sem_o1)

    w = lax.axis_index("s") * NC + lax.axis_index("c")
    node0 = w * NODES_PER_W

    lane = lax.iota(jnp.int32, LANES)
    pat_j = [(lane + v * LANES) % D for v in range(3)]
    pat_i = [3 * ((lane + v * LANES) // D) + (lane + v * LANES) % 3
             for v in range(3)]
    c12 = jnp.full((LANES,), D, jnp.int32)

    def n0_of(ci):
        return node0 + jnp.minimum(ci * CH, TRAILER_START)

    def issue_idx(ci, p):
        pltpu.async_copy(ei_hbm.at[pl.ds(n0_of(ci) * K, ECH)],
                         idx_v[p], sem_i[p])

    def wait_idx(p):
        pltpu.make_async_copy(ei_hbm.at[pl.ds(0, ECH)],
                              idx_v[p], sem_i[p]).wait()

    def issue_gathers(ci, p):
        n0 = n0_of(ci)
        for v in range(CH // LANES):
            idx2_v[p][pl.ds(v * LANES, LANES)] = n0 + lane + v * LANES
        pltpu.async_copy(xm_hbm.at[idx2_v[p]], xi_v[p], sem_g[p])
        pltpu.async_copy(xm_hbm.at[idx_v[p]], rows_v[p], sem_g[p])

    def wait_gathers(p):
        pltpu.make_async_copy(xm_hbm.at[idx2_v[p]], xi_v[p], sem_g[p]).wait()
        pltpu.make_async_copy(xm_hbm.at[idx_v[p]], rows_v[p], sem_g[p]).wait()

    def issue_out(ci, p):
        n0 = n0_of(ci)
        pltpu.async_copy(out_v[p], edge_out.at[pl.ds(n0, CH)], sem_o[p])
        pltpu.async_copy(mask_v[p], mask_out.at[pl.ds(n0, CH)], sem_o[p])
        pltpu.async_copy(node_v[p], node_out.at[pl.ds(n0, CH)], sem_o[p])

    def wait_out(p):
        pltpu.make_async_copy(out_v[p], edge_out.at[pl.ds(0, CH)],
                              sem_o[p]).wait()
        pltpu.make_async_copy(mask_v[p], mask_out.at[pl.ds(0, CH)],
                              sem_o[p]).wait()
        pltpu.make_async_copy(node_v[p], node_out.at[pl.ds(0, CH)],
                              sem_o[p]).wait()

    lane48 = lane * F
    lnmask = lane < D

    def compute(ci, p):
        rows, xi_t, out_t, mask_t = rows_v[p], xi_v[p], out_v[p], mask_v[p]
        node_t = node_v[p]

        def one_node(i):
            e0 = i * K
            spl_i = jnp.full((LANES,), i, jnp.int32)
            erow = e0 + lane
            mi = plsc.load_gather(xi_t, [spl_i, c12])
            mj = plsc.load_gather(rows, [erow, c12])
            m = mi * mj
            mask_t[i, :] = m
            m01 = m * SCALE
            xi_row = plsc.load_gather(xi_t, [spl_i, lane])
            plsc.store_scatter(node_t, [spl_i, lane], (mi * SCALE) * xi_row,
                               mask=lnmask)
            xj = [m01 * plsc.load_gather(
                      rows, [erow, jnp.full((LANES,), c, jnp.int32)])
                  for c in range(D)]
            ti = [m01 * plsc.load_gather(
                      xi_t, [spl_i, jnp.full((LANES,), c, jnp.int32)])
                  for c in range(D)]
            for o in range(F):
                val = xj[o % D] - ti[3 * (o // D) + o % 3]
                plsc.store_scatter(out_t, [spl_i, lane48 + o], val)

        def node_body(t2, carry2):
            one_node(2 * t2)
            one_node(2 * t2 + 1)
            return carry2

        lax.fori_loop(0, CH // 2, node_body, 0)

    issue_idx(0, 0)
    wait_idx(0)
    issue_gathers(0, 0)
    issue_idx(1, 1)

    def pair_body(t, carry):
        for b in range(2):
            ci = 2 * t + b
            wait_gathers(b)

            @pl.when(ci + 1 < N_CHUNKS)
            def _():
                wait_idx(1 - b)
                issue_gathers(ci + 1, 1 - b)

            @pl.when(ci + 2 < N_CHUNKS)
            def _():
                issue_idx(ci + 2, b)

            @pl.when(ci >= 2)
            def _():
                wait_out(b)

            compute(ci, b)
            issue_out(ci, b)
        return carry

    lax.fori_loop(0, N_CHUNKS // 2, pair_body, 0)
    wait_out(0)
    wait_out(1)


def _edge_call(xm, ei):
    mesh = plsc.VectorSubcoreMesh(core_axis_name="c", subcore_axis_name="s")
    f = pl.kernel(
        _edge_body,
        out_type=(
            jax.ShapeDtypeStruct((N, K * F), jnp.float32),
            jax.ShapeDtypeStruct((N, K), jnp.float32),
            jax.ShapeDtypeStruct((N, D), jnp.float32),
        ),
        mesh=mesh,
        compiler_params=pltpu.CompilerParams(
            use_tc_tiling_on_sc=False, needs_layout_passes=False),
        scratch_types=[
            pltpu.VMEM((ECH,), jnp.int32),
            pltpu.VMEM((ECH,), jnp.int32),
            pltpu.VMEM((CH,), jnp.int32),
            pltpu.VMEM((CH,), jnp.int32),
            pltpu.VMEM((CH, 16), jnp.float32),
            pltpu.VMEM((CH, 16), jnp.float32),
            pltpu.VMEM((ECH, 16), jnp.float32),
            pltpu.VMEM((ECH, 16), jnp.float32),
            pltpu.VMEM((CH, K * F), jnp.float32),
            pltpu.VMEM((CH, K * F), jnp.float32),
            pltpu.VMEM((CH, K), jnp.float32),
            pltpu.VMEM((CH, K), jnp.float32),
            pltpu.VMEM((CH, D), jnp.float32),
            pltpu.VMEM((CH, D), jnp.float32),
            pltpu.SemaphoreType.DMA,
            pltpu.SemaphoreType.DMA,
            pltpu.SemaphoreType.DMA,
            pltpu.SemaphoreType.DMA,
            pltpu.SemaphoreType.DMA,
            pltpu.SemaphoreType.DMA,
        ],
    )
    return f(xm, ei)


@jax.jit
def kernel(X, edge_idx, C):
    x_flat = X.reshape(N, D)
    mask = (C.reshape(N) > 0).astype(jnp.float32)
    xm = jnp.concatenate(
        [x_flat, mask[:, None], jnp.zeros((N, 3), jnp.float32)], axis=1)
    ei = edge_idx.reshape(N * K)

    edge_flat, mask_flat, node_h = _edge_call(xm, ei)

    return (node_h.reshape(1, N, D),
            edge_flat.reshape(1, N, K, F),
            mask_flat.reshape(1, N, K, 1))
